# baseline (speedup 1.0000x reference)
import functools

import jax
import jax.numpy as jnp
from jax import lax
from jax.experimental import pallas as pl
from jax.experimental.pallas import tpu as pltpu
from jax.experimental.pallas import tpu_sc as plsc

N = 10000
E = 320000
G = 64
NPAD = 10240
NC, NS = 2, 16
CH = 128
NCHT = 160
EPADT = NS * NCHT * CH
NCH2 = NCHT // 2
KB = 4
NPAIR = NCH2 // (2 * KB)
KB_DEG = 8
RPT = NPAD // NS
SPT = NPAD // NS
SRO = RPT // 8
NB = 1024
GRID = NPAD // NB
FD = 16
F = 32

_mesh = plsc.VectorSubcoreMesh(core_axis_name="c", subcore_axis_name="s")
_sc_params = pltpu.CompilerParams(use_tc_tiling_on_sc=False)


def _run_lap(table_s, acc, src_v, dst_v, rows, gsem0, gsem1, ssem0, ssem1):
    last = NCH2 - KB

    def fire_g(st, j0, sem):
        for b in range(KB):
            pltpu.async_copy(table_s.at[src_v.at[j0 + b]], rows.at[st, b], sem)

    def drain_g(st, j0, sem):
        for b in range(KB):
            pltpu.make_async_copy(
                table_s.at[src_v.at[j0 + b]], rows.at[st, b], sem).wait()

    def fire_s(st, j0, sem):
        for b in range(KB):
            pltpu.async_copy(rows.at[st, b], acc.at[dst_v.at[j0 + b]], sem,
                             add=True)

    def drain_s(st, j0, sem):
        for b in range(KB):
            pltpu.make_async_copy(
                rows.at[st, b], acc.at[dst_v.at[j0 + b]], sem).wait()

    fire_g(0, 0, gsem0)

    def pair(t, carry):
        ja = (2 * t) * KB
        jb = (2 * t + 1) * KB
        jn = jnp.minimum(ja + 2 * KB, last)
        drain_g(0, ja, gsem0)
        fire_s(0, ja, ssem0)
        fire_g(1, jb, gsem1)
        drain_s(0, ja, ssem0)
        drain_g(1, jb, gsem1)
        fire_s(1, jb, ssem1)
        fire_g(0, jn, gsem0)
        drain_s(1, jb, ssem1)
        return carry

    lax.fori_loop(0, NPAIR, pair, 0)
    drain_g(0, last, gsem0)


_LAP2_SCRATCH = [
    pltpu.VMEM_SHARED((NPAD, F), jnp.float32),
    pltpu.VMEM_SHARED((NPAD, F), jnp.float32),
    pltpu.VMEM((NCH2, CH), jnp.int32),
    pltpu.VMEM((NCH2, CH), jnp.int32),
    pltpu.VMEM((2, KB, CH, F), jnp.float32),
    pltpu.SemaphoreType.DMA,
    pltpu.SemaphoreType.DMA,
    pltpu.SemaphoreType.DMA,
    pltpu.SemaphoreType.DMA,
]


@functools.partial(
    pl.kernel,
    out_type=(jax.ShapeDtypeStruct((NC, NPAD, F), jnp.float32),
              jax.ShapeDtypeStruct((NC, NPAD, F), jnp.float32)),
    mesh=_mesh,
    scratch_types=_LAP2_SCRATCH,
    compiler_params=_sc_params,
)
def _lap2_sc(src_hbm, dst_hbm, t1_hbm, t2_hbm, zeros_hbm, out1_hbm, out2_hbm,
             table_s, acc, src_v, dst_v, rows, gsem0, gsem1, ssem0, ssem1):
    c = lax.axis_index("c")
    s = lax.axis_index("s")
    row0 = s * RPT
    ebase = c * NCH2
    pltpu.sync_copy(zeros_hbm.at[pl.ds(row0, RPT)], acc.at[pl.ds(row0, RPT)])
    pltpu.sync_copy(t1_hbm.at[pl.ds(row0, RPT)], table_s.at[pl.ds(row0, RPT)])
    pltpu.sync_copy(src_hbm.at[s, pl.ds(ebase, NCH2)], src_v)
    pltpu.sync_copy(dst_hbm.at[s, pl.ds(ebase, NCH2)], dst_v)
    plsc.subcore_barrier()
    _run_lap(table_s, acc, src_v, dst_v, rows, gsem0, gsem1, ssem0, ssem1)
    plsc.subcore_barrier()
    pltpu.sync_copy(acc.at[pl.ds(row0, RPT)], out1_hbm.at[c, pl.ds(row0, RPT)])
    pltpu.sync_copy(zeros_hbm.at[pl.ds(row0, RPT)], acc.at[pl.ds(row0, RPT)])
    pltpu.sync_copy(t2_hbm.at[pl.ds(row0, RPT)], table_s.at[pl.ds(row0, RPT)])
    plsc.subcore_barrier()
    _run_lap(table_s, acc, src_v, dst_v, rows, gsem0, gsem1, ssem0, ssem1)
    plsc.subcore_barrier()
    pltpu.sync_copy(acc.at[pl.ds(row0, RPT)], out2_hbm.at[c, pl.ds(row0, RPT)])


@functools.partial(
    pl.kernel,
    out_type=jax.ShapeDtypeStruct((NC, NPAD, F), jnp.float32),
    mesh=_mesh,
    scratch_types=_LAP2_SCRATCH + [
        pltpu.VMEM((SRO, F), jnp.float32),
        pltpu.VMEM((SRO, F), jnp.float32),
        pltpu.VMEM((SRO, F), jnp.float32),
    ],
    compiler_params=_sc_params,
)
def _lapc_sc(src_hbm, dst_hbm, ps_hbm, d2_hbm, zeros_hbm, out_hbm,
             table_s, acc, src_v, dst_v, rows, gsem0, gsem1, ssem0, ssem1,
             pa_v, pb_v, d2_v):
    c = lax.axis_index("c")
    s = lax.axis_index("s")
    row0 = s * RPT
    ebase = c * NCH2
    pltpu.sync_copy(zeros_hbm.at[pl.ds(row0, RPT)], acc.at[pl.ds(row0, RPT)])
    pltpu.sync_copy(src_hbm.at[s, pl.ds(ebase, NCH2)], src_v)
    pltpu.sync_copy(dst_hbm.at[s, pl.ds(ebase, NCH2)], dst_v)
    for k in range(RPT // SRO):
        r0 = row0 + k * SRO
        pltpu.sync_copy(ps_hbm.at[0, pl.ds(r0, SRO)], pa_v)
        pltpu.sync_copy(ps_hbm.at[1, pl.ds(r0, SRO)], pb_v)
        pltpu.sync_copy(d2_hbm.at[pl.ds(r0, SRO)], d2_v)

        def mrow(r, carry):
            for c0 in (0, 16):
                a = pa_v[r, pl.ds(c0, 16)]
                b = pb_v[r, pl.ds(c0, 16)]
                d = d2_v[r, pl.ds(c0, 16)]
                pa_v[r, pl.ds(c0, 16)] = (a + b) * d
            return carry

        lax.fori_loop(0, SRO, mrow, 0)
        pltpu.sync_copy(pa_v, table_s.at[pl.ds(r0, SRO)])
    plsc.subcore_barrier()
    _run_lap(table_s, acc, src_v, dst_v, rows, gsem0, gsem1, ssem0, ssem1)
    plsc.subcore_barrier()
    pltpu.sync_copy(acc.at[pl.ds(row0, RPT)], out_hbm.at[c, pl.ds(row0, RPT)])


@functools.partial(
    pl.kernel,
    out_type=jax.ShapeDtypeStruct((NC, NPAD, FD), jnp.float32),
    mesh=_mesh,
    scratch_types=[
        pltpu.VMEM_SHARED((NPAD, FD), jnp.float32),
        pltpu.VMEM((NCHT, CH), jnp.int32),
        pltpu.VMEM((CH, FD), jnp.float32),
        pltpu.SemaphoreType.DMA,
    ],
    compiler_params=_sc_params,
)
def _deg_sc(dst_hbm, zeros_hbm, ones_hbm, out_hbm, acc, dst_v, ones_v, ssem):
    c = lax.axis_index("c")
    s = lax.axis_index("s")
    row0 = s * RPT
    pltpu.sync_copy(zeros_hbm.at[pl.ds(row0, RPT)], acc.at[pl.ds(row0, RPT)])
    pltpu.sync_copy(ones_hbm, ones_v)
    pltpu.sync_copy(dst_hbm.at[s], dst_v)
    plsc.subcore_barrier()
    base = c * (NCHT // 2)

    def group(g, carry):
        j0 = base + g * KB_DEG
        for b in range(KB_DEG):
            pltpu.async_copy(ones_v, acc.at[dst_v.at[j0 + b]], ssem, add=True)
        for b in range(KB_DEG):
            pltpu.make_async_copy(ones_v, acc.at[dst_v.at[j0 + b]], ssem).wait()
        return carry

    lax.fori_loop(0, NCHT // 2 // KB_DEG, group, 0)
    plsc.subcore_barrier()
    pltpu.sync_copy(acc.at[pl.ds(row0, RPT)], out_hbm.at[c, pl.ds(row0, RPT)])


def _dot(a, b):
    return jnp.dot(a, b, preferred_element_type=jnp.float32)


def _tc1_body(p_ref, x_ref, w_ref, dis_ref, d2b_ref, t1_ref, t2_ref,
              u0_ref, u2_ref):
    deg = p_ref[0][:, 0:1] + p_ref[1][:, 0:1]
    dis = jnp.where(deg > 0, lax.rsqrt(jnp.maximum(deg, 1e-12)), 0.0)
    u = _dot(x_ref[...], w_ref[...])
    u2 = u[:, 64:96]
    dis_ref[...] = dis
    d2b_ref[...] = jnp.broadcast_to(dis * dis, (NB, 32))
    t1_ref[...] = dis * u[:, 32:64]
    t2_ref[...] = dis * u2
    u0_ref[...] = u[:, 0:32]
    u2_ref[...] = u2


def _tc3_body(u0_ref, u2_ref, p_ref, q_ref, dis_ref, b_ref,
              w0_ref, w1_ref, w2_ref, t4_ref, t5_ref, v0_ref, v2_ref):
    dis = dis_ref[...]
    s1 = p_ref[0] + p_ref[1]
    s3 = q_ref[0] + q_ref[1]
    h = jax.nn.relu(u0_ref[...] - dis * s1 + 2.0 * dis * s3
                    - u2_ref[...] + b_ref[...])
    v2 = _dot(h, w2_ref[...])
    t4_ref[...] = dis * _dot(h, w1_ref[...])
    t5_ref[...] = dis * v2
    v0_ref[...] = _dot(h, w0_ref[...])
    v2_ref[...] = v2


def _tc5_body(v0_ref, v2_ref, p_ref, q_ref, dis_ref, b_ref, fcw_ref,
              batch_ref, psum_ref, cnt_ref):
    i = pl.program_id(0)
    dis = dis_ref[...]
    s4 = p_ref[0] + p_ref[1]
    s6 = q_ref[0] + q_ref[1]
    h = jax.nn.relu(v0_ref[...] - dis * s4 + 2.0 * dis * s6
                    - v2_ref[...] + b_ref[...])
    r = _dot(h, fcw_ref[...])
    gid = lax.broadcasted_iota(jnp.int32, (G, NB), 0)
    m = (batch_ref[...] == gid).astype(jnp.float32)
    ps = _dot(m, r)
    ct = jnp.sum(m, axis=1, keepdims=True)

    @pl.when(i == 0)
    def _():
        psum_ref[...] = ps
        cnt_ref[...] = ct

    @pl.when(i > 0)
    def _():
        psum_ref[...] = psum_ref[...] + ps
        cnt_ref[...] = cnt_ref[...] + ct


def _tc6_body(psum_ref, cnt_ref, fcb_ref, out_ref):
    out_ref[...] = (psum_ref[...] / jnp.maximum(cnt_ref[...], 1.0)
                    + fcb_ref[...])


def _f32(shape):
    return jax.ShapeDtypeStruct(shape, jnp.float32)


def _row(width):
    return pl.BlockSpec((NB, width), lambda i: (i, 0))


def _prt(width):
    return pl.BlockSpec((2, NB, width), lambda i: (0, i, 0))


def _full(shape):
    return pl.BlockSpec(shape, lambda i: tuple(0 for _ in shape))


_tc1 = pl.pallas_call(
    _tc1_body, grid=(GRID,),
    in_specs=[_prt(FD), _row(128), _full((128, 96))],
    out_specs=(_row(1), _row(32), _row(32), _row(32), _row(32), _row(32)),
    out_shape=(_f32((NPAD, 1)), _f32((NPAD, 32)), _f32((NPAD, 32)),
               _f32((NPAD, 32)), _f32((NPAD, 32)), _f32((NPAD, 32))))
_tc3 = pl.pallas_call(
    _tc3_body, grid=(GRID,),
    in_specs=[_row(32), _row(32), _prt(32), _prt(32), _row(1),
              _full((1, 32)), _full((32, 32)), _full((32, 32)),
              _full((32, 32))],
    out_specs=(_row(32), _row(32), _row(32), _row(32)),
    out_shape=(_f32((NPAD, 32)), _f32((NPAD, 32)), _f32((NPAD, 32)),
               _f32((NPAD, 32))))
_tc5 = pl.pallas_call(
    _tc5_body, grid=(GRID,),
    in_specs=[_row(32), _row(32), _prt(32), _prt(32), _row(1),
              _full((1, 32)), _full((32, 1)),
              pl.BlockSpec((1, NB), lambda i: (0, i))],
    out_specs=(_full((G, 1)), _full((G, 1))),
    out_shape=(_f32((G, 1)), _f32((G, 1))))
_tc6 = pl.pallas_call(
    _tc6_body, out_shape=_f32((G, 1)))


def kernel(x, edge_index, batch, W1, b1, W2, b2, fc_w, fc_b):
    npad_e = EPADT - E
    src = jnp.concatenate(
        [edge_index[0], jnp.zeros((npad_e,), jnp.int32)]).reshape(NS, NCHT, CH)
    dst = jnp.concatenate(
        [edge_index[1], jnp.full((npad_e,), NPAD - 1, jnp.int32)]
    ).reshape(NS, NCHT, CH)
    xp = jnp.concatenate([x, jnp.zeros((NPAD - N, 128), jnp.float32)])
    bp = jnp.concatenate(
        [batch, jnp.full((NPAD - N,), G, jnp.int32)]).reshape(1, NPAD)
    w1all = jnp.concatenate([W1[0], W1[1], W1[2]], axis=1)
    z32 = jnp.zeros((NPAD, F), jnp.float32)
    z16 = jnp.zeros((NPAD, FD), jnp.float32)
    ones16 = jnp.ones((CH, FD), jnp.float32)

    degp = _deg_sc(dst, z16, ones16)
    dis, d2b, t1, t2, u0, u2 = _tc1(degp, xp, w1all)
    ps1, ps2 = _lap2_sc(src, dst, t1, t2, z32)
    ps3 = _lapc_sc(src, dst, ps2, d2b, z32)
    t4, t5, v0, v2 = _tc3(u0, u2, ps1, ps3, dis, b1.reshape(1, 32),
                          W2[0], W2[1], W2[2])
    ps4, ps5 = _lap2_sc(src, dst, t4, t5, z32)
    ps6 = _lapc_sc(src, dst, ps5, d2b, z32)
    psum, cnt = _tc5(v0, v2, ps4, ps6, dis, b2.reshape(1, 32), fc_w, bp)
    out = _tc6(psum, cnt, fc_b.reshape(1, 1))
    return out.reshape(G)


# speedup vs baseline: 1.0093x; 1.0093x over previous
import functools

import jax
import jax.numpy as jnp
from jax import lax
from jax.experimental import pallas as pl
from jax.experimental.pallas import tpu as pltpu
from jax.experimental.pallas import tpu_sc as plsc

N = 10000
E = 320000
G = 64
NPAD = 10240
NC, NS = 2, 16
CH = 128
NCHT = 160
EPADT = NS * NCHT * CH
NCH2 = NCHT // 2
KB = 4
NPAIR = NCH2 // (2 * KB)
KB_DEG = 8
RPT = NPAD // NS
SPT = NPAD // NS
SRO = RPT // 8
NB = 1024
GRID = NPAD // NB
FD = 16
F = 32

_mesh = plsc.VectorSubcoreMesh(core_axis_name="c", subcore_axis_name="s")
_sc_params = pltpu.CompilerParams(use_tc_tiling_on_sc=False)


def _run_lap(table_s, acc, src_v, dst_v, rows, gsem0, gsem1, ssem0, ssem1):
    last = NCH2 - KB

    def fire_g(st, j0, sem):
        for b in range(KB):
            pltpu.async_copy(table_s.at[src_v.at[j0 + b]], rows.at[st, b], sem)

    def drain_g(st, j0, sem):
        for b in range(KB):
            pltpu.make_async_copy(
                table_s.at[src_v.at[j0 + b]], rows.at[st, b], sem).wait()

    def fire_s(st, j0, sem):
        for b in range(KB):
            pltpu.async_copy(rows.at[st, b], acc.at[dst_v.at[j0 + b]], sem,
                             add=True)

    def drain_s(st, j0, sem):
        for b in range(KB):
            pltpu.make_async_copy(
                rows.at[st, b], acc.at[dst_v.at[j0 + b]], sem).wait()

    fire_g(0, 0, gsem0)

    def pair(t, carry):
        ja = (2 * t) * KB
        jb = (2 * t + 1) * KB
        jn = jnp.minimum(ja + 2 * KB, last)
        drain_g(0, ja, gsem0)
        fire_s(0, ja, ssem0)
        fire_g(1, jb, gsem1)
        drain_s(0, ja, ssem0)
        drain_g(1, jb, gsem1)
        fire_s(1, jb, ssem1)
        fire_g(0, jn, gsem0)
        drain_s(1, jb, ssem1)
        return carry

    lax.fori_loop(0, NPAIR, pair, 0)
    drain_g(0, last, gsem0)


_LAP2_SCRATCH = [
    pltpu.VMEM_SHARED((NPAD, F), jnp.float32),
    pltpu.VMEM_SHARED((NPAD, F), jnp.float32),
    pltpu.VMEM((NCH2, CH), jnp.int32),
    pltpu.VMEM((NCH2, CH), jnp.int32),
    pltpu.VMEM((2, KB, CH, F), jnp.float32),
    pltpu.SemaphoreType.DMA,
    pltpu.SemaphoreType.DMA,
    pltpu.SemaphoreType.DMA,
    pltpu.SemaphoreType.DMA,
]


@functools.partial(
    pl.kernel,
    out_type=(jax.ShapeDtypeStruct((NC, NPAD, F), jnp.float32),
              jax.ShapeDtypeStruct((NC, NPAD, F), jnp.float32)),
    mesh=_mesh,
    scratch_types=_LAP2_SCRATCH,
    compiler_params=_sc_params,
)
def _lap2_sc(src_hbm, dst_hbm, t1_hbm, t2_hbm, zeros_hbm, out1_hbm, out2_hbm,
             table_s, acc, src_v, dst_v, rows, gsem0, gsem1, ssem0, ssem1):
    c = lax.axis_index("c")
    s = lax.axis_index("s")
    row0 = s * RPT
    ebase = c * NCH2
    pltpu.sync_copy(zeros_hbm.at[pl.ds(row0, RPT)], acc.at[pl.ds(row0, RPT)])
    pltpu.sync_copy(t1_hbm.at[pl.ds(row0, RPT)], table_s.at[pl.ds(row0, RPT)])
    pltpu.sync_copy(src_hbm.at[s, pl.ds(ebase, NCH2)], src_v)
    pltpu.sync_copy(dst_hbm.at[s, pl.ds(ebase, NCH2)], dst_v)
    plsc.subcore_barrier()
    _run_lap(table_s, acc, src_v, dst_v, rows, gsem0, gsem1, ssem0, ssem1)
    plsc.subcore_barrier()
    pltpu.sync_copy(acc.at[pl.ds(row0, RPT)], out1_hbm.at[c, pl.ds(row0, RPT)])
    pltpu.sync_copy(zeros_hbm.at[pl.ds(row0, RPT)], acc.at[pl.ds(row0, RPT)])
    pltpu.sync_copy(t2_hbm.at[pl.ds(row0, RPT)], table_s.at[pl.ds(row0, RPT)])
    plsc.subcore_barrier()
    _run_lap(table_s, acc, src_v, dst_v, rows, gsem0, gsem1, ssem0, ssem1)
    plsc.subcore_barrier()
    pltpu.sync_copy(acc.at[pl.ds(row0, RPT)], out2_hbm.at[c, pl.ds(row0, RPT)])


@functools.partial(
    pl.kernel,
    out_type=jax.ShapeDtypeStruct((NC, NPAD, F), jnp.float32),
    mesh=_mesh,
    scratch_types=_LAP2_SCRATCH + [
        pltpu.VMEM((SRO, F), jnp.float32),
        pltpu.VMEM((SRO, F), jnp.float32),
        pltpu.VMEM((SRO, F), jnp.float32),
    ],
    compiler_params=_sc_params,
)
def _lapc_sc(src_hbm, dst_hbm, ps_hbm, d2_hbm, zeros_hbm, out_hbm,
             table_s, acc, src_v, dst_v, rows, gsem0, gsem1, ssem0, ssem1,
             pa_v, pb_v, d2_v):
    c = lax.axis_index("c")
    s = lax.axis_index("s")
    row0 = s * RPT
    ebase = c * NCH2
    pltpu.sync_copy(zeros_hbm.at[pl.ds(row0, RPT)], acc.at[pl.ds(row0, RPT)])
    pltpu.sync_copy(src_hbm.at[s, pl.ds(ebase, NCH2)], src_v)
    pltpu.sync_copy(dst_hbm.at[s, pl.ds(ebase, NCH2)], dst_v)
    for k in range(RPT // SRO):
        r0 = row0 + k * SRO
        pltpu.sync_copy(ps_hbm.at[0, pl.ds(r0, SRO)], pa_v)
        pltpu.sync_copy(ps_hbm.at[1, pl.ds(r0, SRO)], pb_v)
        pltpu.sync_copy(d2_hbm.at[pl.ds(r0, SRO)], d2_v)

        def mrow(r4, carry):
            for dr in range(4):
                r = r4 * 4 + dr
                for c0 in (0, 16):
                    a = pa_v[r, pl.ds(c0, 16)]
                    b = pb_v[r, pl.ds(c0, 16)]
                    d = d2_v[r, pl.ds(c0, 16)]
                    pa_v[r, pl.ds(c0, 16)] = (a + b) * d
            return carry

        lax.fori_loop(0, SRO // 4, mrow, 0)
        pltpu.sync_copy(pa_v, table_s.at[pl.ds(r0, SRO)])
    plsc.subcore_barrier()
    _run_lap(table_s, acc, src_v, dst_v, rows, gsem0, gsem1, ssem0, ssem1)
    plsc.subcore_barrier()
    pltpu.sync_copy(acc.at[pl.ds(row0, RPT)], out_hbm.at[c, pl.ds(row0, RPT)])


@functools.partial(
    pl.kernel,
    out_type=jax.ShapeDtypeStruct((NC, NPAD, FD), jnp.float32),
    mesh=_mesh,
    scratch_types=[
        pltpu.VMEM_SHARED((NPAD, FD), jnp.float32),
        pltpu.VMEM((NCHT, CH), jnp.int32),
        pltpu.VMEM((CH, FD), jnp.float32),
        pltpu.SemaphoreType.DMA,
    ],
    compiler_params=_sc_params,
)
def _deg_sc(dst_hbm, zeros_hbm, ones_hbm, out_hbm, acc, dst_v, ones_v, ssem):
    c = lax.axis_index("c")
    s = lax.axis_index("s")
    row0 = s * RPT
    pltpu.sync_copy(zeros_hbm.at[pl.ds(row0, RPT)], acc.at[pl.ds(row0, RPT)])
    pltpu.sync_copy(ones_hbm, ones_v)
    pltpu.sync_copy(dst_hbm.at[s], dst_v)
    plsc.subcore_barrier()
    base = c * (NCHT // 2)

    def group(g, carry):
        j0 = base + g * KB_DEG
        for b in range(KB_DEG):
            pltpu.async_copy(ones_v, acc.at[dst_v.at[j0 + b]], ssem, add=True)
        for b in range(KB_DEG):
            pltpu.make_async_copy(ones_v, acc.at[dst_v.at[j0 + b]], ssem).wait()
        return carry

    lax.fori_loop(0, NCHT // 2 // KB_DEG, group, 0)
    plsc.subcore_barrier()
    pltpu.sync_copy(acc.at[pl.ds(row0, RPT)], out_hbm.at[c, pl.ds(row0, RPT)])


def _dot(a, b):
    return jnp.dot(a, b, preferred_element_type=jnp.float32)


def _tc1_body(p_ref, x_ref, w_ref, dis_ref, d2b_ref, t1_ref, t2_ref,
              u0_ref, u2_ref):
    deg = p_ref[0][:, 0:1] + p_ref[1][:, 0:1]
    dis = jnp.where(deg > 0, lax.rsqrt(jnp.maximum(deg, 1e-12)), 0.0)
    u = _dot(x_ref[...], w_ref[...])
    u2 = u[:, 64:96]
    dis_ref[...] = dis
    d2b_ref[...] = jnp.broadcast_to(dis * dis, (NB, 32))
    t1_ref[...] = dis * u[:, 32:64]
    t2_ref[...] = dis * u2
    u0_ref[...] = u[:, 0:32]
    u2_ref[...] = u2


def _tc3_body(u0_ref, u2_ref, p_ref, q_ref, dis_ref, b_ref,
              w0_ref, w1_ref, w2_ref, t4_ref, t5_ref, v0_ref, v2_ref):
    dis = dis_ref[...]
    s1 = p_ref[0] + p_ref[1]
    s3 = q_ref[0] + q_ref[1]
    h = jax.nn.relu(u0_ref[...] - dis * s1 + 2.0 * dis * s3
                    - u2_ref[...] + b_ref[...])
    v2 = _dot(h, w2_ref[...])
    t4_ref[...] = dis * _dot(h, w1_ref[...])
    t5_ref[...] = dis * v2
    v0_ref[...] = _dot(h, w0_ref[...])
    v2_ref[...] = v2


def _tc5_body(v0_ref, v2_ref, p_ref, q_ref, dis_ref, b_ref, fcw_ref,
              fcb_ref, batch_ref, out_ref, psum_ref, cnt_ref):
    i = pl.program_id(0)
    dis = dis_ref[...]
    s4 = p_ref[0] + p_ref[1]
    s6 = q_ref[0] + q_ref[1]
    h = jax.nn.relu(v0_ref[...] - dis * s4 + 2.0 * dis * s6
                    - v2_ref[...] + b_ref[...])
    r = _dot(h, fcw_ref[...])
    gid = lax.broadcasted_iota(jnp.int32, (G, NB), 0)
    m = (batch_ref[...] == gid).astype(jnp.float32)
    ps = _dot(m, r)
    ct = jnp.sum(m, axis=1, keepdims=True)

    @pl.when(i == 0)
    def _():
        psum_ref[...] = ps
        cnt_ref[...] = ct

    @pl.when(i > 0)
    def _():
        psum_ref[...] = psum_ref[...] + ps
        cnt_ref[...] = cnt_ref[...] + ct

    @pl.when(i == GRID - 1)
    def _():
        out_ref[...] = (psum_ref[...] / jnp.maximum(cnt_ref[...], 1.0)
                        + fcb_ref[...])


def _f32(shape):
    return jax.ShapeDtypeStruct(shape, jnp.float32)


def _row(width):
    return pl.BlockSpec((NB, width), lambda i: (i, 0))


def _prt(width):
    return pl.BlockSpec((2, NB, width), lambda i: (0, i, 0))


def _full(shape):
    return pl.BlockSpec(shape, lambda i: tuple(0 for _ in shape))


_tc1 = pl.pallas_call(
    _tc1_body, grid=(GRID,),
    in_specs=[_prt(FD), _row(128), _full((128, 96))],
    out_specs=(_row(1), _row(32), _row(32), _row(32), _row(32), _row(32)),
    out_shape=(_f32((NPAD, 1)), _f32((NPAD, 32)), _f32((NPAD, 32)),
               _f32((NPAD, 32)), _f32((NPAD, 32)), _f32((NPAD, 32))))
_tc3 = pl.pallas_call(
    _tc3_body, grid=(GRID,),
    in_specs=[_row(32), _row(32), _prt(32), _prt(32), _row(1),
              _full((1, 32)), _full((32, 32)), _full((32, 32)),
              _full((32, 32))],
    out_specs=(_row(32), _row(32), _row(32), _row(32)),
    out_shape=(_f32((NPAD, 32)), _f32((NPAD, 32)), _f32((NPAD, 32)),
               _f32((NPAD, 32))))
_tc5 = pl.pallas_call(
    _tc5_body, grid=(GRID,),
    in_specs=[_row(32), _row(32), _prt(32), _prt(32), _row(1),
              _full((1, 32)), _full((32, 1)), _full((1, 1)),
              pl.BlockSpec((1, NB), lambda i: (0, i))],
    out_specs=_full((G, 1)), out_shape=_f32((G, 1)),
    scratch_shapes=[pltpu.VMEM((G, 1), jnp.float32),
                    pltpu.VMEM((G, 1), jnp.float32)])


def kernel(x, edge_index, batch, W1, b1, W2, b2, fc_w, fc_b):
    npad_e = EPADT - E
    src = jnp.concatenate(
        [edge_index[0], jnp.zeros((npad_e,), jnp.int32)]).reshape(NS, NCHT, CH)
    dst = jnp.concatenate(
        [edge_index[1], jnp.full((npad_e,), NPAD - 1, jnp.int32)]
    ).reshape(NS, NCHT, CH)
    xp = jnp.concatenate([x, jnp.zeros((NPAD - N, 128), jnp.float32)])
    bp = jnp.concatenate(
        [batch, jnp.full((NPAD - N,), G, jnp.int32)]).reshape(1, NPAD)
    w1all = jnp.concatenate([W1[0], W1[1], W1[2]], axis=1)
    z32 = jnp.zeros((NPAD, F), jnp.float32)
    z16 = jnp.zeros((NPAD, FD), jnp.float32)
    ones16 = jnp.ones((CH, FD), jnp.float32)

    degp = _deg_sc(dst, z16, ones16)
    dis, d2b, t1, t2, u0, u2 = _tc1(degp, xp, w1all)
    ps1, ps2 = _lap2_sc(src, dst, t1, t2, z32)
    ps3 = _lapc_sc(src, dst, ps2, d2b, z32)
    t4, t5, v0, v2 = _tc3(u0, u2, ps1, ps3, dis, b1.reshape(1, 32),
                          W2[0], W2[1], W2[2])
    ps4, ps5 = _lap2_sc(src, dst, t4, t5, z32)
    ps6 = _lapc_sc(src, dst, ps5, d2b, z32)
    out = _tc5(v0, v2, ps4, ps6, dis, b2.reshape(1, 32), fc_w,
               fc_b.reshape(1, 1), bp)
    return out.reshape(G)
